# baseline (speedup 1.0000x reference)
import functools

import jax
import jax.numpy as jnp
from jax import lax
from jax.experimental import pallas as pl
from jax.experimental.pallas import tpu as pltpu
from jax.experimental.pallas import tpu_sc as plsc

BLANK = 0
LANES = 16
IDX_MINOR = 128


def _decoder_sc_kernel(b_per_w, n_chunks, num_cores):
    def body(table_hbm, idx_hbm, conv_hbm, scale_hbm, out_hbm,
             idx_v, rows_v, out_v, conv_v, scale_v, sem):
        wid = lax.axis_index("s") * num_cores + lax.axis_index("c")

        pltpu.sync_copy(idx_hbm.at[pl.ds(wid * n_chunks, n_chunks)], idx_v)
        pltpu.sync_copy(conv_hbm, conv_v)
        pltpu.sync_copy(scale_hbm, scale_v)

        copies = [
            pltpu.async_copy(
                table_hbm.at[idx_v.at[c]],
                rows_v.at[pl.ds(c * IDX_MINOR, IDX_MINOR)],
                sem,
            )
            for c in range(n_chunks)
        ]
        for cp in copies:
            cp.wait()

        s = jnp.exp(scale_v[...])
        w0 = [conv_v[0, pl.ds(k * LANES, LANES)] * s for k in range(4)]
        w1 = [conv_v[1, pl.ds(k * LANES, LANES)] * s for k in range(4)]
        zero = jnp.zeros((LANES,), jnp.float32)

        def row_body(n, carry):
            j0 = lax.shift_left(n, 1)
            c0 = lax.shift_right_logical(n, 6)
            l0 = lax.shift_left(jnp.bitwise_and(n, 63), 1)
            cvec = jnp.full((LANES,), c0, jnp.int32)
            i0 = plsc.load_gather(idx_v, [cvec, jnp.full((LANES,), l0, jnp.int32)])
            i1 = plsc.load_gather(idx_v, [cvec, jnp.full((LANES,), l0 + 1, jnp.int32)])
            m0 = i0 != BLANK
            m1 = i1 != BLANK
            for k in range(4):
                v0 = rows_v[j0, pl.ds(k * LANES, LANES)]
                v1 = rows_v[j0 + 1, pl.ds(k * LANES, LANES)]
                acc = (jnp.where(m0, v0, zero) * w0[k]
                       + jnp.where(m1, v1, zero) * w1[k])
                out_v[n, pl.ds(k * LANES, LANES)] = jnp.maximum(acc, zero)
            return carry

        lax.fori_loop(0, b_per_w, row_body, 0, unroll=2)

        pltpu.sync_copy(out_v, out_hbm.at[pl.ds(wid * b_per_w, b_per_w)])

    return body


@jax.jit
def kernel(y, emb_weight, emb_scale, conv_weight, conv_scale):
    batch, ctx = y.shape
    vocab, dim = emb_weight.shape
    assert ctx == 2 and dim == 64

    info = plsc.get_sparse_core_info()
    nw = info.num_cores * info.num_subcores
    b_per_w = batch // nw
    assert batch == nw * b_per_w and (2 * b_per_w) % IDX_MINOR == 0
    n_chunks = (2 * b_per_w) // IDX_MINOR

    idx2d = y.reshape(nw * n_chunks, IDX_MINOR)
    conv2 = jnp.transpose(conv_weight[:, 0, :])
    scale = jnp.full((LANES,), emb_scale + conv_scale, jnp.float32)

    mesh = plsc.VectorSubcoreMesh(core_axis_name="c", subcore_axis_name="s")
    run = functools.partial(
        pl.kernel,
        out_type=jax.ShapeDtypeStruct((batch, dim), jnp.float32),
        mesh=mesh,
        compiler_params=pltpu.CompilerParams(
            needs_layout_passes=False, use_tc_tiling_on_sc=False),
        scratch_types=[
            pltpu.VMEM((n_chunks, IDX_MINOR), jnp.int32),
            pltpu.VMEM((2 * b_per_w, dim), jnp.float32),
            pltpu.VMEM((b_per_w, dim), jnp.float32),
            pltpu.VMEM((2, dim), jnp.float32),
            pltpu.VMEM((LANES,), jnp.float32),
            pltpu.SemaphoreType.DMA,
        ],
    )(_decoder_sc_kernel(b_per_w, n_chunks, info.num_cores))
    out = run(emb_weight, idx2d, conv2, scale)
    return out.reshape(batch, 1, dim)
